# baseline (speedup 1.0000x reference)
import functools

import jax
import jax.numpy as jnp
from jax import lax
from jax.experimental import pallas as pl
from jax.experimental.pallas import tpu as pltpu

M = 100000
D = 768
BM = 5000
NB = M // BM
NOW = 3600.0
NEG_INF = float("-inf")
NSLICE = BM // 128
TAIL = BM - NSLICE * 128


def _insert(ms, mi, v, vi):
    b = [v > m for m in ms]
    new_ms = [jnp.where(b[0], v, ms[0])]
    new_mi = [jnp.where(b[0], vi, mi[0])]
    for j in range(1, 5):
        new_ms.append(jnp.where(b[j - 1], ms[j - 1],
                                jnp.where(b[j], v, ms[j])))
        new_mi.append(jnp.where(b[j - 1], mi[j - 1],
                                jnp.where(b[j], vi, mi[j])))
    return new_ms, new_mi


def _score_topk_kernel(q_ref, loc_ref, feat_ref, mloc_ref, meta_ref,
                       out_s_ref, out_i_ref, qmat_ref, cand_s_ref,
                       cand_i_ref):
    i = pl.program_id(0)
    lane128 = lax.broadcasted_iota(jnp.int32, (1, 128), 1)

    @pl.when(i == 0)
    def _init():
        cand_s_ref[...] = jnp.full((8, 128), NEG_INF, jnp.float32)
        cand_i_ref[...] = jnp.zeros((8, 128), jnp.int32)
        q = q_ref[...]
        qn = q / jnp.maximum(jnp.sqrt(jnp.sum(q * q)), 1e-12)
        qmat_ref[...] = jnp.where(lane128 == 0, jnp.swapaxes(qn, 0, 1), 0.0)

    f = feat_ref[...]
    nsq = jnp.sum(f * f, axis=1, keepdims=True)
    fn = f / jnp.maximum(jnp.sqrt(nsq), 1e-12)
    sim = lax.dot_general(fn, qmat_ref[...], (((1,), (0,)), ((), ())),
                          precision=lax.Precision.DEFAULT)[:, 0:1]

    dl = mloc_ref[...] - loc_ref[...]
    d2 = jnp.sum(dl * dl, axis=1, keepdims=True)
    spatial = 1.0 / (1.0 + jnp.sqrt(d2))

    meta = meta_ref[...]
    strengths = meta[:, 0:1]
    ts = meta[:, 1:2]
    temporal = jnp.exp((ts - NOW) / 3600.0)

    comb_col = (0.5 * sim + 0.3 * spatial + 0.2 * temporal) * strengths
    combined = jnp.swapaxes(comb_col, 0, 1)

    lane = lax.broadcasted_iota(jnp.int32, (1, BM), 1)
    gidx = lane + i * BM

    ms = [cand_s_ref[j:j + 1, :] for j in range(5)]
    mi = [cand_i_ref[j:j + 1, :] for j in range(5)]
    for t in range(NSLICE):
        v = combined[:, t * 128:(t + 1) * 128]
        vi = gidx[:, t * 128:(t + 1) * 128]
        ms, mi = _insert(ms, mi, v, vi)
    pad_s = jnp.full((1, 128 - TAIL), NEG_INF, jnp.float32)
    pad_i = jnp.zeros((1, 128 - TAIL), jnp.int32)
    v = jnp.concatenate([combined[:, NSLICE * 128:], pad_s], axis=1)
    vi = jnp.concatenate([gidx[:, NSLICE * 128:], pad_i], axis=1)
    ms, mi = _insert(ms, mi, v, vi)
    for j in range(5):
        cand_s_ref[j:j + 1, :] = ms[j]
        cand_i_ref[j:j + 1, :] = mi[j]

    @pl.when(i == NB - 1)
    def _final():
        fs = list(ms)
        fi = list(mi)
        os_ = jnp.zeros((1, 128), jnp.float32)
        oi = jnp.zeros((1, 128), jnp.int32)
        for r in range(5):
            t = fs[0]
            for j in range(1, 5):
                t = jnp.maximum(t, fs[j])
            mx = jnp.max(t, axis=1, keepdims=True)
            cm = jnp.full((1, 128), M, jnp.int32)
            for j in range(5):
                cm = jnp.minimum(cm, jnp.where(fs[j] == mx, fi[j], M))
            gi = jnp.min(cm, axis=1, keepdims=True)
            os_ = jnp.where(lane128 == r, mx, os_)
            oi = jnp.where(lane128 == r, gi, oi)
            for j in range(5):
                fs[j] = jnp.where(fi[j] == gi, NEG_INF, fs[j])
        out_s_ref[...] = os_
        out_i_ref[...] = oi


@functools.partial(jax.jit, static_argnames=())
def _run(query_features, location, memory_features, memory_locations,
         memory_metadata):
    q2 = query_features.reshape(1, D)
    loc2 = location.reshape(1, 2)
    out_s, out_i = pl.pallas_call(
        _score_topk_kernel,
        grid=(NB,),
        in_specs=[
            pl.BlockSpec((1, D), lambda i: (0, 0)),
            pl.BlockSpec((1, 2), lambda i: (0, 0)),
            pl.BlockSpec((BM, D), lambda i: (i, 0)),
            pl.BlockSpec((BM, 2), lambda i: (i, 0)),
            pl.BlockSpec((BM, 4), lambda i: (i, 0)),
        ],
        out_specs=[
            pl.BlockSpec((1, 128), lambda i: (0, 0)),
            pl.BlockSpec((1, 128), lambda i: (0, 0)),
        ],
        out_shape=[
            jax.ShapeDtypeStruct((1, 128), jnp.float32),
            jax.ShapeDtypeStruct((1, 128), jnp.int32),
        ],
        scratch_shapes=[
            pltpu.VMEM((D, 128), jnp.float32),
            pltpu.VMEM((8, 128), jnp.float32),
            pltpu.VMEM((8, 128), jnp.int32),
        ],
    )(q2, loc2, memory_features, memory_locations, memory_metadata)
    return out_s[0, :5], out_i[0, :5]


def kernel(query_features, location, memory_features, memory_locations,
           memory_metadata, k):
    del k
    return _run(query_features, location, memory_features,
                memory_locations, memory_metadata)


# speedup vs baseline: 1.0017x; 1.0017x over previous
import functools

import jax
import jax.numpy as jnp
from jax import lax
from jax.experimental import pallas as pl
from jax.experimental.pallas import tpu as pltpu

M = 100000
D = 768
BM = 4000
NB = M // BM
NOW = 3600.0
NEG_INF = float("-inf")
NSLICE = BM // 128
TAIL = BM - NSLICE * 128


def _insert(ms, mi, v, vi):
    b = [v > m for m in ms]
    new_ms = [jnp.where(b[0], v, ms[0])]
    new_mi = [jnp.where(b[0], vi, mi[0])]
    for j in range(1, 5):
        new_ms.append(jnp.where(b[j - 1], ms[j - 1],
                                jnp.where(b[j], v, ms[j])))
        new_mi.append(jnp.where(b[j - 1], mi[j - 1],
                                jnp.where(b[j], vi, mi[j])))
    return new_ms, new_mi


def _score_topk_kernel(q_ref, loc_ref, feat_ref, mloc_ref, meta_ref,
                       out_s_ref, out_i_ref, qmat_ref, cand_s_ref,
                       cand_i_ref):
    i = pl.program_id(0)
    lane128 = lax.broadcasted_iota(jnp.int32, (1, 128), 1)

    @pl.when(i == 0)
    def _init():
        cand_s_ref[...] = jnp.full((8, 128), NEG_INF, jnp.float32)
        cand_i_ref[...] = jnp.zeros((8, 128), jnp.int32)
        q = q_ref[...]
        qn = q / jnp.maximum(jnp.sqrt(jnp.sum(q * q)), 1e-12)
        qmat_ref[...] = jnp.where(lane128 == 0, jnp.swapaxes(qn, 0, 1), 0.0)

    f = feat_ref[...]
    nsq = jnp.sum(f * f, axis=1, keepdims=True)
    fn = f / jnp.maximum(jnp.sqrt(nsq), 1e-12)
    sim = lax.dot_general(fn, qmat_ref[...], (((1,), (0,)), ((), ())),
                          precision=lax.Precision.DEFAULT)[:, 0:1]

    dl = mloc_ref[...] - loc_ref[...]
    d2 = jnp.sum(dl * dl, axis=1, keepdims=True)
    spatial = 1.0 / (1.0 + jnp.sqrt(d2))

    meta = meta_ref[...]
    strengths = meta[:, 0:1]
    ts = meta[:, 1:2]
    temporal = jnp.exp((ts - NOW) / 3600.0)

    comb_col = (0.5 * sim + 0.3 * spatial + 0.2 * temporal) * strengths
    combined = jnp.swapaxes(comb_col, 0, 1)

    lane = lax.broadcasted_iota(jnp.int32, (1, BM), 1)
    gidx = lane + i * BM

    ms = [cand_s_ref[j:j + 1, :] for j in range(5)]
    mi = [cand_i_ref[j:j + 1, :] for j in range(5)]
    for t in range(NSLICE):
        v = combined[:, t * 128:(t + 1) * 128]
        vi = gidx[:, t * 128:(t + 1) * 128]
        ms, mi = _insert(ms, mi, v, vi)
    pad_s = jnp.full((1, 128 - TAIL), NEG_INF, jnp.float32)
    pad_i = jnp.zeros((1, 128 - TAIL), jnp.int32)
    v = jnp.concatenate([combined[:, NSLICE * 128:], pad_s], axis=1)
    vi = jnp.concatenate([gidx[:, NSLICE * 128:], pad_i], axis=1)
    ms, mi = _insert(ms, mi, v, vi)
    for j in range(5):
        cand_s_ref[j:j + 1, :] = ms[j]
        cand_i_ref[j:j + 1, :] = mi[j]

    @pl.when(i == NB - 1)
    def _final():
        fs = list(ms)
        fi = list(mi)
        os_ = jnp.zeros((1, 128), jnp.float32)
        oi = jnp.zeros((1, 128), jnp.int32)
        for r in range(5):
            t = fs[0]
            for j in range(1, 5):
                t = jnp.maximum(t, fs[j])
            mx = jnp.max(t, axis=1, keepdims=True)
            cm = jnp.full((1, 128), M, jnp.int32)
            for j in range(5):
                cm = jnp.minimum(cm, jnp.where(fs[j] == mx, fi[j], M))
            gi = jnp.min(cm, axis=1, keepdims=True)
            os_ = jnp.where(lane128 == r, mx, os_)
            oi = jnp.where(lane128 == r, gi, oi)
            for j in range(5):
                fs[j] = jnp.where(fi[j] == gi, NEG_INF, fs[j])
        out_s_ref[...] = os_
        out_i_ref[...] = oi


@functools.partial(jax.jit, static_argnames=())
def _run(query_features, location, memory_features, memory_locations,
         memory_metadata):
    q2 = query_features.reshape(1, D)
    loc2 = location.reshape(1, 2)
    out_s, out_i = pl.pallas_call(
        _score_topk_kernel,
        grid=(NB,),
        in_specs=[
            pl.BlockSpec((1, D), lambda i: (0, 0)),
            pl.BlockSpec((1, 2), lambda i: (0, 0)),
            pl.BlockSpec((BM, D), lambda i: (i, 0)),
            pl.BlockSpec((BM, 2), lambda i: (i, 0)),
            pl.BlockSpec((BM, 4), lambda i: (i, 0)),
        ],
        out_specs=[
            pl.BlockSpec((1, 128), lambda i: (0, 0)),
            pl.BlockSpec((1, 128), lambda i: (0, 0)),
        ],
        out_shape=[
            jax.ShapeDtypeStruct((1, 128), jnp.float32),
            jax.ShapeDtypeStruct((1, 128), jnp.int32),
        ],
        scratch_shapes=[
            pltpu.VMEM((D, 128), jnp.float32),
            pltpu.VMEM((8, 128), jnp.float32),
            pltpu.VMEM((8, 128), jnp.int32),
        ],
    )(q2, loc2, memory_features, memory_locations, memory_metadata)
    return out_s[0, :5], out_i[0, :5]


def kernel(query_features, location, memory_features, memory_locations,
           memory_metadata, k):
    del k
    return _run(query_features, location, memory_features,
                memory_locations, memory_metadata)
